# baseline (speedup 1.0000x reference)
import functools

import jax
import jax.numpy as jnp
from jax import lax
from jax.experimental import pallas as pl
from jax.experimental.pallas import tpu as pltpu
from jax.experimental.pallas import tpu_sc as plsc

NC = 2
NS = 16
NW = NC * NS
LRELU = 0.2


def _leaky(v):
  return jnp.where(v > 0, v, LRELU * v)


def _proj_body(x_ref, w_ref, b_ref, su2_ref, sv2_ref, b2_ref):
  s = jnp.dot(x_ref[...], w_ref[...].T, preferred_element_type=jnp.float32)
  s = s + b_ref[...]
  su = s[:, :8]
  sv = s[:, 8:]
  su2_ref[...] = jnp.concatenate([su, su], axis=1)
  sv2_ref[...] = jnp.concatenate([sv, sv], axis=1)
  m = jnp.max(s, axis=0, keepdims=True)
  bb = _leaky(m[:, :8] + m[:, 8:])
  b2_ref[...] = jnp.concatenate([bb, bb], axis=1)


def _make_k2(n, npad, e, b, nchunk, rows):
  mesh = plsc.VectorSubcoreMesh(core_axis_name="c", subcore_axis_name="s")

  @functools.partial(
      pl.kernel,
      out_type=(
          jax.ShapeDtypeStruct((npad, 16), jnp.float32),
          jax.ShapeDtypeStruct((npad, 16), jnp.float32),
          jax.ShapeDtypeStruct((npad, 128), jnp.float32),
          jax.ShapeDtypeStruct((npad, 128), jnp.float32),
      ),
      mesh=mesh,
      compiler_params=pltpu.CompilerParams(
          use_tc_tiling_on_sc=False, needs_layout_passes=False),
      scratch_types=[
          pltpu.VMEM((b,), jnp.int32),
          pltpu.VMEM((b,), jnp.int32),
          pltpu.VMEM((b,), jnp.int32),
          pltpu.VMEM((b,), jnp.int32),
          pltpu.VMEM((b,), jnp.int32),
          pltpu.VMEM((b,), jnp.int32),
          pltpu.VMEM((b,), jnp.int32),
          pltpu.VMEM((b,), jnp.int32),
          pltpu.VMEM((b,), jnp.int32),
          pltpu.VMEM((b, 16), jnp.float32),
          pltpu.VMEM((b, 16), jnp.float32),
          pltpu.VMEM((b, 16), jnp.float32),
          pltpu.VMEM((b, 160), jnp.bfloat16),
          pltpu.VMEM((b, 160), jnp.bfloat16),
          pltpu.VMEM((b, 160), jnp.bfloat16),
          pltpu.VMEM((b, 144), jnp.float32),
          pltpu.VMEM((b, 144), jnp.float32),
          pltpu.VMEM((b, 144), jnp.float32),
          pltpu.VMEM((16,), jnp.float32),
          pltpu.VMEM_SHARED((npad, 144), jnp.float32),
          pltpu.SemaphoreType.DMA,
          pltpu.SemaphoreType.DMA,
          pltpu.SemaphoreType.DMA,
          pltpu.SemaphoreType.DMA,
          pltpu.SemaphoreType.DMA,
          pltpu.SemaphoreType.DMA,
          pltpu.SemaphoreType.DMA,
          pltpu.SemaphoreType.DMA,
          pltpu.SemaphoreType.DMA,
      ],
  )
  def k2(xs_h, sv2_h, b2_h, ei_h, z144_h,
         da_h, db_h, pa_h, pb_h,
         srcv0, srcv1, srcv2, dstv0, dstv1, dstv2, sdst0, sdst1, sdst2,
         svb0, svb1, svb2,
         xb0, xb1, xb2, mb0, mb1, mb2, bnd_v, acc,
         semi0, semi1, semi2, semg0, semg1, semg2, sems0, sems1, sems2):
    cid = lax.axis_index("c")
    sid = lax.axis_index("s")
    tbase = (cid * NS + sid) * (nchunk * b)
    sl = pl.ds(sid * rows, rows)
    pltpu.sync_copy(z144_h.at[sl], acc.at[sl])
    pltpu.sync_copy(b2_h.at[0], bnd_v)
    plsc.subcore_barrier()

    srcv = (srcv0, srcv1, srcv2)
    dstv = (dstv0, dstv1, dstv2)
    sdst = (sdst0, sdst1, sdst2)
    svb = (svb0, svb1, svb2)
    xb = (xb0, xb1, xb2)
    mb = (mb0, mb1, mb2)
    semi = (semi0, semi1, semi2)
    semg = (semg0, semg1, semg2)
    sems = (sems0, sems1, sems2)

    def idx_issue(t, r):
      base = tbase + t * b
      pltpu.async_copy(ei_h.at[0, pl.ds(base, b)], srcv[r], semi[r])
      pltpu.async_copy(ei_h.at[1, pl.ds(base, b)], dstv[r], semi[r])

    def idx_wait(t, r):
      base = tbase + t * b
      pltpu.make_async_copy(ei_h.at[0, pl.ds(base, b)], srcv[r], semi[r]).wait()
      pltpu.make_async_copy(ei_h.at[1, pl.ds(base, b)], dstv[r], semi[r]).wait()

    def gather_issue(t, r):
      pltpu.async_copy(sv2_h.at[dstv[r]], svb[r], semg[r])
      pltpu.async_copy(xs_h.at[srcv[r]], xb[r], semg[r])

    def gather_wait(t, r):
      pltpu.make_async_copy(sv2_h.at[dstv[r]], svb[r], semg[r]).wait()
      pltpu.make_async_copy(xs_h.at[srcv[r]], xb[r], semg[r]).wait()

    def scatter_drain(t, r):
      pltpu.make_async_copy(mb[r], acc.at[sdst[r]], sems[r]).wait()

    def compute_scatter(t, r):
      bnd = bnd_v[...]
      for k in range(b):
        vsu = xb[r][k, pl.ds(128, 32)]
        shi, slo = plsc.unpack(vsu, format=plsc.PackFormat.INTERLEAVED)
        ev = _leaky((shi + slo) + svb[r][k])
        exv = jnp.exp(ev - bnd)
        mb[r][k, pl.ds(128, 16)] = exv
        for c in range(4):
          v32 = xb[r][k, pl.ds(32 * c, 32)]
          lo, hi = plsc.unpack(v32, format=plsc.PackFormat.INTERLEAVED)
          mb[r][k, pl.ds(32 * c, 16)] = lo * exv
          mb[r][k, pl.ds(32 * c + 16, 16)] = hi * exv
      for q in range(b // 16):
        sdst[r][pl.ds(16 * q, 16)] = dstv[r][pl.ds(16 * q, 16)]
      if b % 16:
        sdst[r][pl.ds(b - 16, 16)] = dstv[r][pl.ds(b - 16, 16)]
      pltpu.async_copy(mb[r], acc.at[sdst[r]], sems[r], add=True)

    idx_issue(0, 0)
    idx_wait(0, 0)
    gather_issue(0, 0)
    idx_issue(1, 1)
    idx_wait(1, 1)
    gather_issue(1, 1)
    idx_issue(2, 2)

    nloop = nchunk // 3

    @pl.loop(0, nloop)
    def _triple(j):
      t0 = 3 * j
      for u in range(3):
        t = t0 + u
        r = u

        @pl.when(t >= 2)
        def _(t=t, u=u):
          scatter_drain(t - 2, (u + 1) % 3)

        @pl.when(t + 2 < nchunk)
        def _(t=t, u=u):
          idx_wait(t + 2, (u + 2) % 3)
          gather_issue(t + 2, (u + 2) % 3)

        gather_wait(t, r)
        compute_scatter(t, r)

        @pl.when(t + 3 < nchunk)
        def _(t=t, u=u):
          idx_issue(t + 3, u)

    for t in range(3 * nloop, nchunk):
      r = t % 3
      if t >= 2:
        scatter_drain(t - 2, (t - 2) % 3)
      if t + 2 < nchunk:
        idx_wait(t + 2, (t + 2) % 3)
        gather_issue(t + 2, (t + 2) % 3)
      gather_wait(t, r)
      compute_scatter(t, r)
      if t + 3 < nchunk:
        idx_issue(t + 3, (t + 3) % 3)
    scatter_drain(nchunk - 2, (nchunk - 2) % 3)
    scatter_drain(nchunk - 1, (nchunk - 1) % 3)

    plsc.subcore_barrier()

    @pl.when(cid == 0)
    def _():
      pltpu.sync_copy(acc.at[sl, pl.ds(128, 16)], da_h.at[sl])
      pltpu.sync_copy(acc.at[sl, pl.ds(0, 128)], pa_h.at[sl])

    @pl.when(cid == 1)
    def _():
      pltpu.sync_copy(acc.at[sl, pl.ds(128, 16)], db_h.at[sl])
      pltpu.sync_copy(acc.at[sl, pl.ds(0, 128)], pb_h.at[sl])

  return k2


def _norm_body(x_ref, da_ref, db_ref, pa_ref, pb_ref, o_ref):
  o_ref[:, :128] = x_ref[...]
  dsum = da_ref[...] + db_ref[...] + 1e-16
  den = jnp.concatenate([dsum] * 8, axis=1)
  o_ref[:, 128:] = (pa_ref[...] + pb_ref[...]) / den


def kernel(x, edge_index, Wu, bu, Wv):
  n, d = x.shape
  e = edge_index.shape[1]
  ept = e // NW
  b = 40
  nchunk = ept // b
  npad = ((n + NS * 8 - 1) // (NS * 8)) * NS * 8
  rows = npad // NS


  w_all = jnp.concatenate([Wu, Wv], axis=0)
  b16 = jnp.concatenate([bu, jnp.zeros((8,), jnp.float32)])[None, :]

  su2, sv2, b2 = pl.pallas_call(
      _proj_body,
      out_shape=(
          jax.ShapeDtypeStruct((n, 16), jnp.float32),
          jax.ShapeDtypeStruct((n, 16), jnp.float32),
          jax.ShapeDtypeStruct((1, 16), jnp.float32),
      ),
  )(x, w_all, b16)

  xr = x.reshape(n, 4, 2, 16)
  xs = jnp.stack([xr[:, :, 0, :], xr[:, :, 1, :]], axis=-1)
  xs = xs.reshape(n, 128).astype(jnp.bfloat16)
  su_hi = su2.astype(jnp.bfloat16)
  su_lo = (su2 - su_hi.astype(jnp.float32)).astype(jnp.bfloat16)
  sui = jnp.stack([su_hi, su_lo], axis=-1).reshape(n, 32)
  xs = jnp.concatenate([xs, sui], axis=1)

  z144 = jnp.zeros((npad, 144), jnp.float32)
  da, db, pa, pb = _make_k2(n, npad, e, b, nchunk, rows)(
      xs, sv2, b2, edge_index, z144)

  blk = 1000
  out = pl.pallas_call(
      _norm_body,
      grid=(n // blk,),
      in_specs=[
          pl.BlockSpec((blk, d), lambda i: (i, 0)),
          pl.BlockSpec((blk, 16), lambda i: (i, 0)),
          pl.BlockSpec((blk, 16), lambda i: (i, 0)),
          pl.BlockSpec((blk, d), lambda i: (i, 0)),
          pl.BlockSpec((blk, d), lambda i: (i, 0)),
      ],
      out_specs=pl.BlockSpec((blk, 2 * d), lambda i: (i, 0)),
      out_shape=jax.ShapeDtypeStruct((n, 2 * d), jnp.float32),
  )(x, da, db, pa, pb)
  return out


# speedup vs baseline: 1.1369x; 1.1369x over previous
import functools

import jax
import jax.numpy as jnp
from jax import lax
from jax.experimental import pallas as pl
from jax.experimental.pallas import tpu as pltpu
from jax.experimental.pallas import tpu_sc as plsc

NC = 2
NS = 16
NW = NC * NS
LRELU = 0.2


def _leaky(v):
  return jnp.where(v > 0, v, LRELU * v)


def _proj_body(x_ref, w_ref, b_ref, su2_ref, sv2_ref, b2_ref):
  s = jnp.dot(x_ref[...], w_ref[...].T, preferred_element_type=jnp.float32)
  s = s + b_ref[...]
  su = s[:, :8]
  sv = s[:, 8:]
  su2_ref[...] = jnp.concatenate([su, su], axis=1)
  sv2_ref[...] = jnp.concatenate([sv, sv], axis=1)
  m = jnp.max(s, axis=0, keepdims=True)
  bb = _leaky(m[:, :8] + m[:, 8:])
  b2_ref[...] = jnp.concatenate([bb, bb], axis=1)


def _make_k2(n, npad, e, b, nchunk, rows):
  mesh = plsc.VectorSubcoreMesh(core_axis_name="c", subcore_axis_name="s")

  @functools.partial(
      pl.kernel,
      out_type=(
          jax.ShapeDtypeStruct((npad, 16), jnp.float32),
          jax.ShapeDtypeStruct((npad, 16), jnp.float32),
          jax.ShapeDtypeStruct((npad, 128), jnp.float32),
          jax.ShapeDtypeStruct((npad, 128), jnp.float32),
      ),
      mesh=mesh,
      compiler_params=pltpu.CompilerParams(
          use_tc_tiling_on_sc=False, needs_layout_passes=False),
      scratch_types=[
          pltpu.VMEM((b,), jnp.int32),
          pltpu.VMEM((b,), jnp.int32),
          pltpu.VMEM((b,), jnp.int32),
          pltpu.VMEM((b,), jnp.int32),
          pltpu.VMEM((b,), jnp.int32),
          pltpu.VMEM((b,), jnp.int32),
          pltpu.VMEM((b,), jnp.int32),
          pltpu.VMEM((b,), jnp.int32),
          pltpu.VMEM((b,), jnp.int32),
          pltpu.VMEM((b, 16), jnp.float32),
          pltpu.VMEM((b, 16), jnp.float32),
          pltpu.VMEM((b, 16), jnp.float32),
          pltpu.VMEM((b, 16), jnp.float32),
          pltpu.VMEM((b, 16), jnp.float32),
          pltpu.VMEM((b, 16), jnp.float32),
          pltpu.VMEM((b, 128), jnp.bfloat16),
          pltpu.VMEM((b, 128), jnp.bfloat16),
          pltpu.VMEM((b, 128), jnp.bfloat16),
          pltpu.VMEM((b, 144), jnp.float32),
          pltpu.VMEM((b, 144), jnp.float32),
          pltpu.VMEM((b, 144), jnp.float32),
          pltpu.VMEM((16,), jnp.float32),
          pltpu.VMEM_SHARED((npad, 144), jnp.float32),
          pltpu.SemaphoreType.DMA,
          pltpu.SemaphoreType.DMA,
          pltpu.SemaphoreType.DMA,
          pltpu.SemaphoreType.DMA,
          pltpu.SemaphoreType.DMA,
          pltpu.SemaphoreType.DMA,
          pltpu.SemaphoreType.DMA,
          pltpu.SemaphoreType.DMA,
          pltpu.SemaphoreType.DMA,
      ],
  )
  def k2(xs_h, su2_h, sv2_h, b2_h, ei_h, z144_h,
         da_h, db_h, pa_h, pb_h,
         srcv0, srcv1, srcv2, dstv0, dstv1, dstv2, sdst0, sdst1, sdst2,
         sub0, sub1, sub2, svb0, svb1, svb2,
         xb0, xb1, xb2, mb0, mb1, mb2, bnd_v, acc,
         semi0, semi1, semi2, semg0, semg1, semg2, sems0, sems1, sems2):
    cid = lax.axis_index("c")
    sid = lax.axis_index("s")
    tbase = (cid * NS + sid) * (nchunk * b)
    sl = pl.ds(sid * rows, rows)
    pltpu.sync_copy(z144_h.at[sl], acc.at[sl])
    pltpu.sync_copy(b2_h.at[0], bnd_v)
    plsc.subcore_barrier()

    srcv = (srcv0, srcv1, srcv2)
    dstv = (dstv0, dstv1, dstv2)
    sdst = (sdst0, sdst1, sdst2)
    sub = (sub0, sub1, sub2)
    svb = (svb0, svb1, svb2)
    xb = (xb0, xb1, xb2)
    mb = (mb0, mb1, mb2)
    semi = (semi0, semi1, semi2)
    semg = (semg0, semg1, semg2)
    sems = (sems0, sems1, sems2)

    def idx_issue(t, r):
      base = tbase + t * b
      pltpu.async_copy(ei_h.at[0, pl.ds(base, b)], srcv[r], semi[r])
      pltpu.async_copy(ei_h.at[1, pl.ds(base, b)], dstv[r], semi[r])

    def idx_wait(t, r):
      base = tbase + t * b
      pltpu.make_async_copy(ei_h.at[0, pl.ds(base, b)], srcv[r], semi[r]).wait()
      pltpu.make_async_copy(ei_h.at[1, pl.ds(base, b)], dstv[r], semi[r]).wait()

    def gather_issue(t, r):
      pltpu.async_copy(su2_h.at[srcv[r]], sub[r], semg[r])
      pltpu.async_copy(sv2_h.at[dstv[r]], svb[r], semg[r])
      pltpu.async_copy(xs_h.at[srcv[r]], xb[r], semg[r])

    def gather_wait(t, r):
      pltpu.make_async_copy(su2_h.at[srcv[r]], sub[r], semg[r]).wait()
      pltpu.make_async_copy(sv2_h.at[dstv[r]], svb[r], semg[r]).wait()
      pltpu.make_async_copy(xs_h.at[srcv[r]], xb[r], semg[r]).wait()

    def scatter_drain(t, r):
      pltpu.make_async_copy(mb[r], acc.at[sdst[r]], sems[r]).wait()

    def compute_scatter(t, r):
      bnd = bnd_v[...]
      for k in range(b):
        ev = _leaky(sub[r][k] + svb[r][k])
        exv = jnp.exp(ev - bnd)
        mb[r][k, pl.ds(128, 16)] = exv
        for c in range(4):
          v32 = xb[r][k, pl.ds(32 * c, 32)]
          lo, hi = plsc.unpack(v32, format=plsc.PackFormat.INTERLEAVED)
          mb[r][k, pl.ds(32 * c, 16)] = lo * exv
          mb[r][k, pl.ds(32 * c + 16, 16)] = hi * exv
      for q in range(b // 16):
        sdst[r][pl.ds(16 * q, 16)] = dstv[r][pl.ds(16 * q, 16)]
      if b % 16:
        sdst[r][pl.ds(b - 16, 16)] = dstv[r][pl.ds(b - 16, 16)]
      pltpu.async_copy(mb[r], acc.at[sdst[r]], sems[r], add=True)

    idx_issue(0, 0)
    idx_wait(0, 0)
    gather_issue(0, 0)
    idx_issue(1, 1)
    idx_wait(1, 1)
    gather_issue(1, 1)
    idx_issue(2, 2)

    nloop = nchunk // 3

    @pl.loop(0, nloop)
    def _triple(j):
      t0 = 3 * j
      for u in range(3):
        t = t0 + u
        r = u

        @pl.when(t >= 2)
        def _(t=t, u=u):
          scatter_drain(t - 2, (u + 1) % 3)

        @pl.when(t + 2 < nchunk)
        def _(t=t, u=u):
          idx_wait(t + 2, (u + 2) % 3)
          gather_issue(t + 2, (u + 2) % 3)

        gather_wait(t, r)
        compute_scatter(t, r)

        @pl.when(t + 3 < nchunk)
        def _(t=t, u=u):
          idx_issue(t + 3, u)

    for t in range(3 * nloop, nchunk):
      r = t % 3
      if t >= 2:
        scatter_drain(t - 2, (t - 2) % 3)
      if t + 2 < nchunk:
        idx_wait(t + 2, (t + 2) % 3)
        gather_issue(t + 2, (t + 2) % 3)
      gather_wait(t, r)
      compute_scatter(t, r)
      if t + 3 < nchunk:
        idx_issue(t + 3, (t + 3) % 3)
    scatter_drain(nchunk - 2, (nchunk - 2) % 3)
    scatter_drain(nchunk - 1, (nchunk - 1) % 3)

    plsc.subcore_barrier()

    @pl.when(cid == 0)
    def _():
      pltpu.sync_copy(acc.at[sl, pl.ds(128, 16)], da_h.at[sl])
      pltpu.sync_copy(acc.at[sl, pl.ds(0, 128)], pa_h.at[sl])

    @pl.when(cid == 1)
    def _():
      pltpu.sync_copy(acc.at[sl, pl.ds(128, 16)], db_h.at[sl])
      pltpu.sync_copy(acc.at[sl, pl.ds(0, 128)], pb_h.at[sl])

  return k2


def _norm_body(x_ref, da_ref, db_ref, pa_ref, pb_ref, o_ref):
  o_ref[:, :128] = x_ref[...]
  dsum = da_ref[...] + db_ref[...] + 1e-16
  den = jnp.concatenate([dsum] * 8, axis=1)
  o_ref[:, 128:] = (pa_ref[...] + pb_ref[...]) / den


def kernel(x, edge_index, Wu, bu, Wv):
  n, d = x.shape
  e = edge_index.shape[1]
  ept = e // NW
  b = 40
  nchunk = ept // b
  npad = ((n + NS * 8 - 1) // (NS * 8)) * NS * 8
  rows = npad // NS


  w_all = jnp.concatenate([Wu, Wv], axis=0)
  b16 = jnp.concatenate([bu, jnp.zeros((8,), jnp.float32)])[None, :]

  su2, sv2, b2 = pl.pallas_call(
      _proj_body,
      out_shape=(
          jax.ShapeDtypeStruct((n, 16), jnp.float32),
          jax.ShapeDtypeStruct((n, 16), jnp.float32),
          jax.ShapeDtypeStruct((1, 16), jnp.float32),
      ),
  )(x, w_all, b16)

  xr = x.reshape(n, 4, 2, 16)
  xs = jnp.stack([xr[:, :, 0, :], xr[:, :, 1, :]], axis=-1)
  xs = xs.reshape(n, 128).astype(jnp.bfloat16)

  z144 = jnp.zeros((npad, 144), jnp.float32)
  da, db, pa, pb = _make_k2(n, npad, e, b, nchunk, rows)(
      xs, su2, sv2, b2, edge_index, z144)

  blk = 1000
  out = pl.pallas_call(
      _norm_body,
      grid=(n // blk,),
      in_specs=[
          pl.BlockSpec((blk, d), lambda i: (i, 0)),
          pl.BlockSpec((blk, 16), lambda i: (i, 0)),
          pl.BlockSpec((blk, 16), lambda i: (i, 0)),
          pl.BlockSpec((blk, d), lambda i: (i, 0)),
          pl.BlockSpec((blk, d), lambda i: (i, 0)),
      ],
      out_specs=pl.BlockSpec((blk, 2 * d), lambda i: (i, 0)),
      out_shape=jax.ShapeDtypeStruct((n, 2 * d), jnp.float32),
  )(x, da, db, pa, pb)
  return out
